# baseline (speedup 1.0000x reference)
import functools

import jax
import jax.numpy as jnp
from jax import lax
from jax.experimental import pallas as pl
from jax.experimental.pallas import tpu as pltpu
from jax.experimental.pallas import tpu_sc as plsc

_T = 2048
_H = 1024
_E = 8
_K = 2
_DFF = 1024

_TMS = 256
_NTS = (_T * _K) // _TMS + _E
_NSLOT = _NTS * _TMS
_NW = 32
_NEG = -1e30


def _min_index(mask, idx):
    return jnp.min(jnp.where(mask, idx, 10**9), axis=1, keepdims=True)


def _route_book_body(scores_ref, bias_ref, pos_ref, w_ref, pe_ref):
    scores = scores_ref[...]
    s_choice = scores + bias_ref[...]
    iota_e = lax.broadcasted_iota(jnp.int32, (_T, _E), 1)

    left = jnp.concatenate([s_choice[:, 1:], s_choice[:, :1]], axis=1)
    right = jnp.concatenate([s_choice[:, -1:], s_choice[:, :-1]], axis=1)
    gsum = s_choice + jnp.where(iota_e % 2 == 0, left, right)

    m1 = jnp.max(gsum, axis=1, keepdims=True)
    a1 = _min_index(gsum == m1, iota_e) // 2
    gs2 = jnp.where(iota_e // 2 == a1, _NEG, gsum)
    m2 = jnp.max(gs2, axis=1, keepdims=True)
    a2 = _min_index(gs2 == m2, iota_e) // 2
    grp_ok = (iota_e // 2 == a1) | (iota_e // 2 == a2)

    masked = jnp.where(grp_ok, s_choice, _NEG)
    e_m1 = jnp.max(masked, axis=1, keepdims=True)
    e_i1 = _min_index(masked == e_m1, iota_e)
    masked2 = jnp.where(iota_e == e_i1, _NEG, masked)
    e_m2 = jnp.max(masked2, axis=1, keepdims=True)
    e_i2 = _min_index(masked2 == e_m2, iota_e)

    w1 = jnp.sum(jnp.where(iota_e == e_i1, scores, 0.0), axis=1, keepdims=True)
    w2 = jnp.sum(jnp.where(iota_e == e_i2, scores, 0.0), axis=1, keepdims=True)
    wsum = w1 + w2 + 1e-20
    w1n, w2n = w1 / wsum, w2 / wsum

    e_lo = jnp.minimum(e_i1, e_i2)
    e_hi = jnp.maximum(e_i1, e_i2)
    w_lo = jnp.where(e_i1 < e_i2, w1n, w2n)
    w_hi = jnp.where(e_i1 < e_i2, w2n, w1n)
    mask = (iota_e == e_lo) | (iota_e == e_hi)
    mask_bf = mask.astype(jnp.bfloat16)

    tr = lax.broadcasted_iota(jnp.int32, (_T, _T), 0)
    tc = lax.broadcasted_iota(jnp.int32, (_T, _T), 1)
    LT = (tc < tr).astype(jnp.bfloat16)
    prefix = lax.dot_general(LT, mask_bf, (((1,), (0,)), ((), ())),
                             preferred_element_type=jnp.float32)

    cnt = jnp.sum(mask.astype(jnp.float32), axis=0, keepdims=True).astype(jnp.int32)
    pc = ((cnt + (_TMS - 1)) // _TMS) * _TMS
    incl = pc
    for s in (1, 2, 4):
        incl = incl + jnp.concatenate(
            [jnp.zeros((1, s), jnp.int32), incl[:, :-s]], axis=1)
    excl = incl - pc

    position = prefix.astype(jnp.int32) + excl
    pos_lo = jnp.sum(jnp.where(iota_e == e_lo, position, 0), axis=1, keepdims=True)
    pos_hi = jnp.sum(jnp.where(iota_e == e_hi, position, 0), axis=1, keepdims=True)

    pos_ref[...] = jnp.concatenate([pos_lo, pos_hi], axis=1)
    w_ref[...] = jnp.concatenate([w_lo, w_hi], axis=1)
    pe_ref[...] = jnp.concatenate(
        [excl, incl, cnt, jnp.zeros((1, _E), jnp.int32)], axis=1)


def _route_book(scores, bias):
    return pl.pallas_call(
        _route_book_body,
        out_shape=[
            jax.ShapeDtypeStruct((_T, _K), jnp.int32),
            jax.ShapeDtypeStruct((_T, _K), jnp.float32),
            jax.ShapeDtypeStruct((1, 32), jnp.int32),
        ],
    )(scores, bias.reshape(1, _E))


_APW = (_T * _K) // _NW
_CH = 32


def _sc_dispatch(hs, pos_flat):
    mesh = plsc.VectorSubcoreMesh(core_axis_name="c", subcore_axis_name="s")

    @functools.partial(
        pl.kernel, mesh=mesh,
        out_type=jax.ShapeDtypeStruct((_NSLOT, _H), jnp.float32),
        scratch_types=[pltpu.VMEM((_CH,), jnp.int32),
                       pltpu.VMEM((_CH, _H), jnp.float32),
                       pltpu.SemaphoreType.DMA],
    )
    def k(hs_hbm, pos_hbm, out_hbm, idxv, rowsv, sem):
        wid = lax.axis_index("s") * 2 + lax.axis_index("c")
        ebase = wid * _APW
        tbase = (wid % 16) * _APW
        for c in range(_APW // _CH):
            pltpu.sync_copy(pos_hbm.at[pl.ds(ebase + c * _CH, _CH)], idxv)
            pltpu.sync_copy(hs_hbm.at[pl.ds(tbase + c * _CH, _CH)], rowsv)
            pltpu.async_copy(rowsv, out_hbm.at[idxv], sem).wait()

    return k(hs, pos_flat)


def _sc_combine(rows, pos_flat):
    nrows = _T * _K
    rpw = nrows // _NW
    mesh = plsc.VectorSubcoreMesh(core_axis_name="c", subcore_axis_name="s")

    @functools.partial(
        pl.kernel, mesh=mesh,
        out_type=jax.ShapeDtypeStruct((nrows, _H), jnp.float32),
        scratch_types=[pltpu.VMEM((_CH,), jnp.int32),
                       pltpu.VMEM((_CH, _H), jnp.float32),
                       pltpu.SemaphoreType.DMA],
    )
    def k(rows_hbm, pos_hbm, out_hbm, idxv, rowsv, sem):
        wid = lax.axis_index("s") * 2 + lax.axis_index("c")
        base = wid * rpw
        for c in range(rpw // _CH):
            off = base + c * _CH
            pltpu.sync_copy(pos_hbm.at[pl.ds(off, _CH)], idxv)
            pltpu.async_copy(rows_hbm.at[idxv], rowsv, sem).wait()
            pltpu.sync_copy(rowsv, out_hbm.at[pl.ds(off, _CH)])

    return k(rows, pos_flat)


def _silu_mul(gu):
    g = gu[:, :_DFF]
    u = gu[:, _DFF:]
    return (g * jax.nn.sigmoid(g)) * u


def _gmm_body(pe_ref, x_ref, wgu_ref, wd_ref, out_ref):
    i = pl.program_id(0)
    total = pe_ref[15]

    @pl.when(i * _TMS < total)
    def _():
        x = x_ref[...].astype(jnp.bfloat16)
        gu = lax.dot_general(x, wgu_ref[0], (((1,), (1,)), ((), ())),
                             preferred_element_type=jnp.float32)
        act = _silu_mul(gu).astype(jnp.bfloat16)
        out_ref[...] = lax.dot_general(act, wd_ref[0], (((1,), (1,)), ((), ())),
                                       preferred_element_type=jnp.float32)


def _expert_of(i, pe_ref):
    te = jnp.int32(0)
    for e in range(_E):
        te = te + jnp.where(i * _TMS >= pe_ref[8 + e], 1, 0).astype(jnp.int32)
    return jnp.minimum(te, _E - 1)


def _grouped_mm(pe16, x_sorted, wgu, wd):
    grid_spec = pltpu.PrefetchScalarGridSpec(
        num_scalar_prefetch=1,
        grid=(_NTS,),
        in_specs=[
            pl.BlockSpec((_TMS, _H), lambda i, pe: (i, 0)),
            pl.BlockSpec((1, 2 * _DFF, _H), lambda i, pe: (_expert_of(i, pe), 0, 0)),
            pl.BlockSpec((1, _H, _DFF), lambda i, pe: (_expert_of(i, pe), 0, 0)),
        ],
        out_specs=pl.BlockSpec((_TMS, _H), lambda i, pe: (i, 0)),
    )
    return pl.pallas_call(
        _gmm_body,
        grid_spec=grid_spec,
        out_shape=jax.ShapeDtypeStruct((_NSLOT, _H), jnp.float32),
    )(pe16, x_sorted, wgu, wd)


_TMC = 256


def _shared_ffn_body(hs_ref, swgu_ref, swd_ref, out_ref):
    x = hs_ref[...]
    sgu = lax.dot_general(x, swgu_ref[...], (((1,), (1,)), ((), ())),
                          preferred_element_type=jnp.float32)
    out_ref[...] = lax.dot_general(_silu_mul(sgu).astype(jnp.bfloat16),
                                   swd_ref[...], (((1,), (1,)), ((), ())),
                                   preferred_element_type=jnp.float32)


def _shared_ffn(hs_bf, swgu, swd):
    nt = _T // _TMC
    return pl.pallas_call(
        _shared_ffn_body,
        grid=(nt,),
        in_specs=[
            pl.BlockSpec((_TMC, _H), lambda t: (t, 0)),
            pl.BlockSpec((2 * _DFF, _H), lambda t: (0, 0)),
            pl.BlockSpec((_H, _DFF), lambda t: (0, 0)),
        ],
        out_specs=pl.BlockSpec((_TMC, _H), lambda t: (t, 0)),
        out_shape=jax.ShapeDtypeStruct((_T, _H), jnp.float32),
    )(hs_bf, swgu, swd)


def _final_comb_body(sh_ref, g0_ref, g1_ref, w0_ref, w1_ref, out_ref):
    out_ref[...] = (sh_ref[...] + g0_ref[...] * w0_ref[...]
                    + g1_ref[...] * w1_ref[...])


def _final_comb(sh, rows_g, w0, w1):
    nt = _T // _TMC
    return pl.pallas_call(
        _final_comb_body,
        grid=(nt,),
        in_specs=[
            pl.BlockSpec((_TMC, _H), lambda t: (t, 0)),
            pl.BlockSpec((_TMC, _H), lambda t: (t, 0)),
            pl.BlockSpec((_TMC, _H), lambda t: (t + _T // _TMC, 0)),
            pl.BlockSpec((_TMC, 1), lambda t: (t, 0)),
            pl.BlockSpec((_TMC, 1), lambda t: (t, 0)),
        ],
        out_specs=pl.BlockSpec((_TMC, _H), lambda t: (t, 0)),
        out_shape=jax.ShapeDtypeStruct((_T, _H), jnp.float32),
    )(sh, rows_g, rows_g, w0, w1)


def kernel(hidden_states, gate_weight, e_score_correction_bias, w_gate_up,
           w_down, shared_w_gate_up, shared_w_down):
    bf = jnp.bfloat16
    scores = jax.nn.sigmoid((hidden_states @ gate_weight.T).astype(jnp.float32))

    posw, ww, pe32 = _route_book(scores, e_score_correction_bias)
    pos_flat = posw.T.reshape(_T * _K)
    pe_flat = pe32.reshape(32)

    x_sorted = _sc_dispatch(hidden_states, pos_flat)
    sh = _shared_ffn(hidden_states.astype(bf), shared_w_gate_up.astype(bf),
                     shared_w_down.astype(bf))
    rows = _grouped_mm(pe_flat[:16], x_sorted, w_gate_up.astype(bf),
                       w_down.astype(bf))
    rows_g = _sc_combine(rows, pos_flat)
    return _final_comb(sh, rows_g, ww[:, :1], ww[:, 1:])


# speedup vs baseline: 1.0577x; 1.0577x over previous
import functools

import jax
import jax.numpy as jnp
from jax import lax
from jax.experimental import pallas as pl
from jax.experimental.pallas import tpu as pltpu
from jax.experimental.pallas import tpu_sc as plsc

_T = 2048
_H = 1024
_E = 8
_K = 2
_DFF = 1024

_TMS = 256
_NTS = (_T * _K) // _TMS + _E
_NSLOT = _NTS * _TMS
_NW = 32
_NEG = -1e30


def _min_index(mask, idx):
    return jnp.min(jnp.where(mask, idx, 10**9), axis=1, keepdims=True)


def _route_book_body(scores_ref, bias_ref, pos_ref, w_ref, pe_ref):
    scores = scores_ref[...]
    s_choice = scores + bias_ref[...]
    iota_e = lax.broadcasted_iota(jnp.int32, (_T, _E), 1)

    left = jnp.concatenate([s_choice[:, 1:], s_choice[:, :1]], axis=1)
    right = jnp.concatenate([s_choice[:, -1:], s_choice[:, :-1]], axis=1)
    gsum = s_choice + jnp.where(iota_e % 2 == 0, left, right)

    m1 = jnp.max(gsum, axis=1, keepdims=True)
    a1 = _min_index(gsum == m1, iota_e) // 2
    gs2 = jnp.where(iota_e // 2 == a1, _NEG, gsum)
    m2 = jnp.max(gs2, axis=1, keepdims=True)
    a2 = _min_index(gs2 == m2, iota_e) // 2
    grp_ok = (iota_e // 2 == a1) | (iota_e // 2 == a2)

    masked = jnp.where(grp_ok, s_choice, _NEG)
    e_m1 = jnp.max(masked, axis=1, keepdims=True)
    e_i1 = _min_index(masked == e_m1, iota_e)
    masked2 = jnp.where(iota_e == e_i1, _NEG, masked)
    e_m2 = jnp.max(masked2, axis=1, keepdims=True)
    e_i2 = _min_index(masked2 == e_m2, iota_e)

    w1 = jnp.sum(jnp.where(iota_e == e_i1, scores, 0.0), axis=1, keepdims=True)
    w2 = jnp.sum(jnp.where(iota_e == e_i2, scores, 0.0), axis=1, keepdims=True)
    wsum = w1 + w2 + 1e-20
    w1n, w2n = w1 / wsum, w2 / wsum

    e_lo = jnp.minimum(e_i1, e_i2)
    e_hi = jnp.maximum(e_i1, e_i2)
    w_lo = jnp.where(e_i1 < e_i2, w1n, w2n)
    w_hi = jnp.where(e_i1 < e_i2, w2n, w1n)
    mask = (iota_e == e_lo) | (iota_e == e_hi)
    mask_bf = mask.astype(jnp.bfloat16)

    tr = lax.broadcasted_iota(jnp.int32, (_T, _T), 0)
    tc = lax.broadcasted_iota(jnp.int32, (_T, _T), 1)
    LT = (tc < tr).astype(jnp.bfloat16)
    prefix = lax.dot_general(LT, mask_bf, (((1,), (0,)), ((), ())),
                             preferred_element_type=jnp.float32)

    cnt = jnp.sum(mask.astype(jnp.float32), axis=0, keepdims=True).astype(jnp.int32)
    pc = ((cnt + (_TMS - 1)) // _TMS) * _TMS
    incl = pc
    for s in (1, 2, 4):
        incl = incl + jnp.concatenate(
            [jnp.zeros((1, s), jnp.int32), incl[:, :-s]], axis=1)
    excl = incl - pc

    position = prefix.astype(jnp.int32) + excl
    pos_lo = jnp.sum(jnp.where(iota_e == e_lo, position, 0), axis=1, keepdims=True)
    pos_hi = jnp.sum(jnp.where(iota_e == e_hi, position, 0), axis=1, keepdims=True)

    pos_ref[...] = jnp.concatenate([pos_lo, pos_hi], axis=1)
    w_ref[...] = jnp.concatenate([w_lo, w_hi], axis=1)
    pe_ref[...] = jnp.concatenate(
        [excl, incl, cnt, jnp.zeros((1, _E), jnp.int32)], axis=1)


def _route_book(scores, bias):
    return pl.pallas_call(
        _route_book_body,
        out_shape=[
            jax.ShapeDtypeStruct((_T, _K), jnp.int32),
            jax.ShapeDtypeStruct((_T, _K), jnp.float32),
            jax.ShapeDtypeStruct((1, 32), jnp.int32),
        ],
    )(scores, bias.reshape(1, _E))


_APW = (_T * _K) // _NW
_CH = 32


def _sc_dispatch(hs, pos_flat):
    mesh = plsc.VectorSubcoreMesh(core_axis_name="c", subcore_axis_name="s")

    @functools.partial(
        pl.kernel, mesh=mesh,
        out_type=jax.ShapeDtypeStruct((_NSLOT, _H), jnp.float32),
        scratch_types=[pltpu.VMEM((_CH,), jnp.int32),
                       pltpu.VMEM((_CH, _H), jnp.float32),
                       pltpu.SemaphoreType.DMA],
    )
    def k(hs_hbm, pos_hbm, out_hbm, idxv, rowsv, sem):
        wid = lax.axis_index("s") * 2 + lax.axis_index("c")
        ebase = wid * _APW
        tbase = (wid % 16) * _APW
        for c in range(_APW // _CH):
            pltpu.sync_copy(pos_hbm.at[pl.ds(ebase + c * _CH, _CH)], idxv)
            pltpu.sync_copy(hs_hbm.at[pl.ds(tbase + c * _CH, _CH)], rowsv)
            pltpu.async_copy(rowsv, out_hbm.at[idxv], sem).wait()

    return k(hs, pos_flat)


def _sc_combine(rows, pos_flat):
    nrows = _T * _K
    rpw = nrows // _NW
    mesh = plsc.VectorSubcoreMesh(core_axis_name="c", subcore_axis_name="s")

    @functools.partial(
        pl.kernel, mesh=mesh,
        out_type=jax.ShapeDtypeStruct((nrows, _H), jnp.float32),
        scratch_types=[pltpu.VMEM((_CH,), jnp.int32),
                       pltpu.VMEM((_CH, _H), jnp.float32),
                       pltpu.SemaphoreType.DMA],
    )
    def k(rows_hbm, pos_hbm, out_hbm, idxv, rowsv, sem):
        wid = lax.axis_index("s") * 2 + lax.axis_index("c")
        base = wid * rpw
        for c in range(rpw // _CH):
            off = base + c * _CH
            pltpu.sync_copy(pos_hbm.at[pl.ds(off, _CH)], idxv)
            pltpu.async_copy(rows_hbm.at[idxv], rowsv, sem).wait()
            pltpu.sync_copy(rowsv, out_hbm.at[pl.ds(off, _CH)])

    return k(rows, pos_flat)


def _silu_mul(gu):
    g = gu[:, :_DFF]
    u = gu[:, _DFF:]
    return (g * jax.nn.sigmoid(g)) * u


def _gmm_body(pe_ref, x_ref, wgu_ref, wd_ref, out_ref):
    i = pl.program_id(0)
    total = pe_ref[15]

    @pl.when(i * _TMS < total)
    def _():
        x = x_ref[...].astype(jnp.bfloat16)
        gu = lax.dot_general(x, wgu_ref[0], (((1,), (1,)), ((), ())),
                             preferred_element_type=jnp.float32)
        act = _silu_mul(gu).astype(jnp.bfloat16)
        out_ref[...] = lax.dot_general(act, wd_ref[0], (((1,), (1,)), ((), ())),
                                       preferred_element_type=jnp.float32)


def _expert_of(i, pe_ref):
    te = jnp.int32(0)
    for e in range(_E):
        te = te + jnp.where(i * _TMS >= pe_ref[8 + e], 1, 0).astype(jnp.int32)
    return jnp.minimum(te, _E - 1)


def _grouped_mm(pe16, x_sorted, wgu, wd):
    grid_spec = pltpu.PrefetchScalarGridSpec(
        num_scalar_prefetch=1,
        grid=(_NTS,),
        in_specs=[
            pl.BlockSpec((_TMS, _H), lambda i, pe: (i, 0)),
            pl.BlockSpec((1, 2 * _DFF, _H), lambda i, pe: (_expert_of(i, pe), 0, 0)),
            pl.BlockSpec((1, _H, _DFF), lambda i, pe: (_expert_of(i, pe), 0, 0)),
        ],
        out_specs=pl.BlockSpec((_TMS, _H), lambda i, pe: (i, 0)),
    )
    return pl.pallas_call(
        _gmm_body,
        grid_spec=grid_spec,
        out_shape=jax.ShapeDtypeStruct((_NSLOT, _H), jnp.float32),
    )(pe16, x_sorted, wgu, wd)


_TMC = 256


def _shared_comb_body(hs_ref, swgu_ref, swd_ref, g0_ref, g1_ref,
                      w0_ref, w1_ref, out_ref):
    x = hs_ref[...]
    sgu = lax.dot_general(x, swgu_ref[...], (((1,), (1,)), ((), ())),
                          preferred_element_type=jnp.float32)
    sout = lax.dot_general(_silu_mul(sgu).astype(jnp.bfloat16), swd_ref[...],
                           (((1,), (1,)), ((), ())),
                           preferred_element_type=jnp.float32)
    out_ref[...] = (sout + g0_ref[...] * w0_ref[...]
                    + g1_ref[...] * w1_ref[...])


def _shared_comb(hs_bf, swgu, swd, rows_g, w0, w1):
    nt = _T // _TMC
    return pl.pallas_call(
        _shared_comb_body,
        grid=(nt,),
        in_specs=[
            pl.BlockSpec((_TMC, _H), lambda t: (t, 0)),
            pl.BlockSpec((2 * _DFF, _H), lambda t: (0, 0)),
            pl.BlockSpec((_H, _DFF), lambda t: (0, 0)),
            pl.BlockSpec((_TMC, _H), lambda t: (t, 0)),
            pl.BlockSpec((_TMC, _H), lambda t: (t + _T // _TMC, 0)),
            pl.BlockSpec((_TMC, 1), lambda t: (t, 0)),
            pl.BlockSpec((_TMC, 1), lambda t: (t, 0)),
        ],
        out_specs=pl.BlockSpec((_TMC, _H), lambda t: (t, 0)),
        out_shape=jax.ShapeDtypeStruct((_T, _H), jnp.float32),
    )(hs_bf, swgu, swd, rows_g, rows_g, w0, w1)


def kernel(hidden_states, gate_weight, e_score_correction_bias, w_gate_up,
           w_down, shared_w_gate_up, shared_w_down):
    bf = jnp.bfloat16
    scores = jax.nn.sigmoid((hidden_states @ gate_weight.T).astype(jnp.float32))

    posw, ww, pe32 = _route_book(scores, e_score_correction_bias)
    pos_flat = posw.T.reshape(_T * _K)
    pe_flat = pe32.reshape(32)

    x_sorted = _sc_dispatch(hidden_states, pos_flat)
    rows = _grouped_mm(pe_flat[:16], x_sorted, w_gate_up.astype(bf),
                       w_down.astype(bf))
    rows_g = _sc_combine(rows, pos_flat)
    return _shared_comb(hidden_states.astype(bf), shared_w_gate_up.astype(bf),
                        shared_w_down.astype(bf), rows_g,
                        ww[:, :1], ww[:, 1:])
